# baseline (device time: 368265 ns/iter reference)
import jax
import jax.numpy as jnp
from jax import lax
from jax.experimental import pallas as pl
from jax.experimental.pallas import tpu as pltpu

_N_CH = 16
_CLIP = 6.0


def _fused_exchange_softmax(logits, q8):
    t, v = logits.shape
    rows = t // _N_CH

    def body(
        logits_ref,
        q_ref,
        final_ref,
        commq_ref,
        lbufs,
        rbufs,
        olbufs,
        orbufs,
        send_sem,
        recv_sems,
        lsems, rsems,
        osems,
    ):
        my_x = lax.axis_index("x")
        my_y = lax.axis_index("y")
        my_z = lax.axis_index("z")
        nbr = (my_x, 1 - my_y, my_z)

        barrier_sem = pltpu.get_barrier_semaphore()
        pl.semaphore_signal(
            barrier_sem, inc=1, device_id=nbr,
            device_id_type=pl.DeviceIdType.MESH,
        )
        pl.semaphore_wait(barrier_sem, 1)

        rdmas = []
        for i in range(_N_CH):
            rs = pl.ds(i * rows, rows)
            r = pltpu.make_async_remote_copy(
                src_ref=q_ref.at[rs],
                dst_ref=commq_ref.at[rs],
                send_sem=send_sem,
                recv_sem=recv_sems.at[i],
                device_id=nbr,
                device_id_type=pl.DeviceIdType.MESH,
            )
            r.start()
            rdmas.append(r)

        stores = []
        for i in range(_N_CH):
            s = i % 2
            rs = pl.ds(i * rows, rows)
            if i >= 2:
                stores[i - 2][0].wait()
                stores[i - 2][1].wait()
            cp_l = pltpu.make_async_copy(
                logits_ref.at[rs], lbufs.at[s], lsems.at[s]
            )
            cp_l.start()
            rdmas[i].wait_recv()
            cp_r = pltpu.make_async_copy(
                commq_ref.at[rs], rbufs.at[s], rsems.at[s]
            )
            cp_r.start()
            cp_l.wait()
            cp_r.wait()

            l = lbufs[s]
            r = rbufs[s].astype(jnp.float32) * (_CLIP / 127.0)
            e_l = jnp.exp(l)
            e_r = jnp.exp(r)
            inv = 1.0 / (
                jnp.sum(e_l, axis=1, keepdims=True)
                + jnp.sum(e_r, axis=1, keepdims=True)
            )
            olbufs[s] = e_l * inv
            orbufs[s] = e_r * inv

            st_l = pltpu.make_async_copy(
                olbufs.at[s], final_ref.at[rs, pl.ds(my_y * v, v)], osems.at[s]
            )
            st_l.start()
            st_r = pltpu.make_async_copy(
                orbufs.at[s],
                final_ref.at[rs, pl.ds((1 - my_y) * v, v)],
                osems.at[s],
            )
            st_r.start()
            stores.append((st_l, st_r))

        for st_l, st_r in stores[-2:]:
            st_l.wait()
            st_r.wait()
        for r in rdmas:
            r.wait_send()

    final, _ = pl.pallas_call(
        body,
        out_shape=(
            jax.ShapeDtypeStruct((t, 2 * v), jnp.float32),
            jax.ShapeDtypeStruct((t, v), jnp.int8),
        ),
        in_specs=[
            pl.BlockSpec(memory_space=pl.ANY),
            pl.BlockSpec(memory_space=pl.ANY),
        ],
        out_specs=(
            pl.BlockSpec(memory_space=pl.ANY),
            pl.BlockSpec(memory_space=pl.ANY),
        ),
        scratch_shapes=[
            pltpu.VMEM((2, rows, v), jnp.float32),
            pltpu.VMEM((2, rows, v), jnp.int8),
            pltpu.VMEM((2, rows, v), jnp.float32),
            pltpu.VMEM((2, rows, v), jnp.float32),
            pltpu.SemaphoreType.DMA,
            pltpu.SemaphoreType.DMA((_N_CH,)),
            pltpu.SemaphoreType.DMA((2,)),
            pltpu.SemaphoreType.DMA((2,)),
            pltpu.SemaphoreType.DMA((2,)),
        ],
        compiler_params=pltpu.CompilerParams(
            collective_id=0, vmem_limit_bytes=48 * 1024 * 1024
        ),
    )(logits, q8)
    return final


def kernel(x, W):
    logits = jnp.dot(x, W, preferred_element_type=jnp.float32)
    q8 = jnp.round(
        jnp.clip(logits, -_CLIP, _CLIP) * (127.0 / _CLIP)
    ).astype(jnp.int8)
    return _fused_exchange_softmax(logits, q8)


# device time: 363096 ns/iter; 1.0142x vs baseline; 1.0142x over previous
import jax
import jax.numpy as jnp
from jax import lax
from jax.experimental import pallas as pl
from jax.experimental.pallas import tpu as pltpu

_N_CH = 32
_CLIP = 6.0


def _fused_exchange_softmax(logits, q8):
    t, v = logits.shape
    rows = t // _N_CH

    def body(
        logits_ref,
        q_ref,
        final_ref,
        commq_ref,
        lbufs,
        rbufs,
        olbufs,
        orbufs,
        send_sem,
        recv_sems,
        lsems, rsems,
        osems,
    ):
        my_x = lax.axis_index("x")
        my_y = lax.axis_index("y")
        my_z = lax.axis_index("z")
        nbr = (my_x, 1 - my_y, my_z)

        barrier_sem = pltpu.get_barrier_semaphore()
        pl.semaphore_signal(
            barrier_sem, inc=1, device_id=nbr,
            device_id_type=pl.DeviceIdType.MESH,
        )
        pl.semaphore_wait(barrier_sem, 1)

        rdmas = []
        for i in range(_N_CH):
            rs = pl.ds(i * rows, rows)
            r = pltpu.make_async_remote_copy(
                src_ref=q_ref.at[rs],
                dst_ref=commq_ref.at[rs],
                send_sem=send_sem,
                recv_sem=recv_sems.at[i],
                device_id=nbr,
                device_id_type=pl.DeviceIdType.MESH,
            )
            r.start()
            rdmas.append(r)

        stores = []
        for i in range(_N_CH):
            s = i % 2
            rs = pl.ds(i * rows, rows)
            if i >= 2:
                stores[i - 2][0].wait()
                stores[i - 2][1].wait()
            cp_l = pltpu.make_async_copy(
                logits_ref.at[rs], lbufs.at[s], lsems.at[s]
            )
            cp_l.start()
            rdmas[i].wait_recv()
            cp_r = pltpu.make_async_copy(
                commq_ref.at[rs], rbufs.at[s], rsems.at[s]
            )
            cp_r.start()
            cp_l.wait()
            cp_r.wait()

            l = lbufs[s]
            r = rbufs[s].astype(jnp.float32) * (_CLIP / 127.0)
            e_l = jnp.exp(l)
            e_r = jnp.exp(r)
            inv = 1.0 / (
                jnp.sum(e_l, axis=1, keepdims=True)
                + jnp.sum(e_r, axis=1, keepdims=True)
            )
            olbufs[s] = e_l * inv
            orbufs[s] = e_r * inv

            st_l = pltpu.make_async_copy(
                olbufs.at[s], final_ref.at[rs, pl.ds(my_y * v, v)], osems.at[s]
            )
            st_l.start()
            st_r = pltpu.make_async_copy(
                orbufs.at[s],
                final_ref.at[rs, pl.ds((1 - my_y) * v, v)],
                osems.at[s],
            )
            st_r.start()
            stores.append((st_l, st_r))

        for st_l, st_r in stores[-2:]:
            st_l.wait()
            st_r.wait()
        for r in rdmas:
            r.wait_send()

    final, _ = pl.pallas_call(
        body,
        out_shape=(
            jax.ShapeDtypeStruct((t, 2 * v), jnp.float32),
            jax.ShapeDtypeStruct((t, v), jnp.int8),
        ),
        in_specs=[
            pl.BlockSpec(memory_space=pl.ANY),
            pl.BlockSpec(memory_space=pl.ANY),
        ],
        out_specs=(
            pl.BlockSpec(memory_space=pl.ANY),
            pl.BlockSpec(memory_space=pl.ANY),
        ),
        scratch_shapes=[
            pltpu.VMEM((2, rows, v), jnp.float32),
            pltpu.VMEM((2, rows, v), jnp.int8),
            pltpu.VMEM((2, rows, v), jnp.float32),
            pltpu.VMEM((2, rows, v), jnp.float32),
            pltpu.SemaphoreType.DMA,
            pltpu.SemaphoreType.DMA((_N_CH,)),
            pltpu.SemaphoreType.DMA((2,)),
            pltpu.SemaphoreType.DMA((2,)),
            pltpu.SemaphoreType.DMA((2,)),
        ],
        compiler_params=pltpu.CompilerParams(
            collective_id=0, vmem_limit_bytes=48 * 1024 * 1024
        ),
    )(logits, q8)
    return final


def kernel(x, W):
    logits = jnp.dot(x, W, preferred_element_type=jnp.float32)
    q8 = jnp.round(
        jnp.clip(logits, -_CLIP, _CLIP) * (127.0 / _CLIP)
    ).astype(jnp.int8)
    return _fused_exchange_softmax(logits, q8)
